# Initial kernel scaffold; baseline (speedup 1.0000x reference)
import jax
import jax.numpy as jnp
from jax.experimental import pallas as pl


def kernel(x, W):
    raise NotImplementedError("write your pallas kernel here")


# speedup vs baseline: 2.9769x; 2.9769x over previous
import functools

import jax
import jax.numpy as jnp
from jax import lax
from jax.experimental import pallas as pl
from jax.experimental.pallas import tpu as pltpu
from jax.experimental.pallas import tpu_sc as plsc

D_MODEL = 128
B_TOTAL = 4096 * 50
NUM_CORES = 2
NUM_SUBCORES = 16
NW = NUM_CORES * NUM_SUBCORES
BPW = B_TOTAL // NW
CHUNK = 128
NCHUNK = BPW // CHUNK

_mesh = plsc.VectorSubcoreMesh(core_axis_name="c", subcore_axis_name="s")


@functools.partial(
    pl.kernel,
    mesh=_mesh,
    out_type=jax.ShapeDtypeStruct((B_TOTAL, D_MODEL), jnp.float32),
    scratch_types=[
        pltpu.VMEM((NCHUNK, CHUNK), jnp.int32),
        pltpu.VMEM((CHUNK, D_MODEL), jnp.float32),
        pltpu.SemaphoreType.DMA,
    ],
)
def _embed_sc(x_hbm, w_hbm, out_hbm, idx_v, rows_v, sem):
    wid = lax.axis_index("s") * NUM_CORES + lax.axis_index("c")
    base = wid * BPW
    pltpu.sync_copy(x_hbm.at[wid], idx_v)

    def body(c, carry):
        pltpu.async_copy(w_hbm.at[idx_v.at[c]], rows_v, sem).wait()
        pltpu.sync_copy(rows_v, out_hbm.at[pl.ds(base + c * CHUNK, CHUNK)])
        return carry

    lax.fori_loop(0, NCHUNK, body, 0)


def kernel(x, W):
    x_blocks = x.reshape(NW, NCHUNK, CHUNK).astype(jnp.int32)
    out = _embed_sc(x_blocks, W)
    return out.reshape(x.shape[0], x.shape[1], D_MODEL)


# speedup vs baseline: 3.3503x; 1.1254x over previous
import functools

import jax
import jax.numpy as jnp
from jax import lax
from jax.experimental import pallas as pl
from jax.experimental.pallas import tpu as pltpu
from jax.experimental.pallas import tpu_sc as plsc

D_MODEL = 128
B_TOTAL = 4096 * 50
NUM_CORES = 2
NUM_SUBCORES = 16
NW = NUM_CORES * NUM_SUBCORES
BPW = B_TOTAL // NW
CHUNK = 128
NCHUNK = BPW // CHUNK
NB = 5

_mesh = plsc.VectorSubcoreMesh(core_axis_name="c", subcore_axis_name="s")


@functools.partial(
    pl.kernel,
    mesh=_mesh,
    out_type=jax.ShapeDtypeStruct((B_TOTAL, D_MODEL), jnp.float32),
    scratch_types=[
        pltpu.VMEM((NCHUNK, CHUNK), jnp.int32),
        pltpu.VMEM((NB, CHUNK, D_MODEL), jnp.float32),
        pltpu.SemaphoreType.DMA((NB,)),
    ],
)
def _embed_sc(x_hbm, w_hbm, out_hbm, idx_v, rows_v, sems):
    wid = lax.axis_index("s") * NUM_CORES + lax.axis_index("c")
    base = wid * BPW
    pltpu.sync_copy(x_hbm.at[wid], idx_v)

    def gather(c, b):
        return pltpu.make_async_copy(
            w_hbm.at[idx_v.at[c]], rows_v.at[b], sems.at[b])

    for b in range(NB):
        gather(b, b).start()

    def body(t, carry):
        c0 = t * NB
        for b in range(NB):
            c = c0 + b
            gather(c, b).wait()
            pltpu.sync_copy(rows_v.at[b],
                            out_hbm.at[pl.ds(base + c * CHUNK, CHUNK)])

            @pl.when(c + NB < NCHUNK)
            def _():
                gather(c + NB, b).start()

        return carry

    lax.fori_loop(0, NCHUNK // NB, body, 0)


def kernel(x, W):
    x_blocks = x.reshape(NW, NCHUNK, CHUNK).astype(jnp.int32)
    out = _embed_sc(x_blocks, W)
    return out.reshape(x.shape[0], x.shape[1], D_MODEL)


# speedup vs baseline: 5.7745x; 1.7236x over previous
import functools

import jax
import jax.numpy as jnp
from jax import lax
from jax.experimental import pallas as pl
from jax.experimental.pallas import tpu as pltpu
from jax.experimental.pallas import tpu_sc as plsc

D_MODEL = 128
SEQ = 50
BATCH = 4096
B_TOTAL = BATCH * SEQ
NUM_CORES = 2
NUM_SUBCORES = 16
NW = NUM_CORES * NUM_SUBCORES
EPW = BATCH // NW
SEQ_PAD = 56
BPW = EPW * SEQ_PAD
NB = 8

_mesh = plsc.VectorSubcoreMesh(core_axis_name="c", subcore_axis_name="s")


@functools.partial(
    pl.kernel,
    mesh=_mesh,
    out_type=jax.ShapeDtypeStruct((BATCH, SEQ, D_MODEL), jnp.float32),
    scratch_types=[
        pltpu.VMEM((BPW,), jnp.int32),
        pltpu.VMEM((NB, SEQ_PAD, D_MODEL), jnp.float32),
        pltpu.SemaphoreType.DMA((NB,)),
    ],
    compiler_params=pltpu.CompilerParams(use_tc_tiling_on_sc=True),
)
def _embed_sc(x_hbm, w_hbm, out_hbm, idx_v, rows_v, sems):
    wid = lax.axis_index("s") * NUM_CORES + lax.axis_index("c")
    ebase = wid * EPW
    pltpu.sync_copy(x_hbm.at[pl.ds(wid * BPW, BPW)], idx_v)

    def gather(e, b):
        return pltpu.make_async_copy(
            w_hbm.at[idx_v.at[pl.ds(e * SEQ_PAD, SEQ_PAD)]],
            rows_v.at[b], sems.at[b])

    for b in range(NB):
        gather(b, b).start()

    def body(t, carry):
        e0 = t * NB
        for b in range(NB):
            e = e0 + b
            gather(e, b).wait()
            pltpu.sync_copy(rows_v.at[b, pl.ds(0, SEQ)], out_hbm.at[ebase + e])

            @pl.when(e + NB < EPW)
            def _():
                gather(e + NB, b).start()

        return carry

    lax.fori_loop(0, EPW // NB, body, 0)


def kernel(x, W):
    xi = x.astype(jnp.int32)
    x_pad = jnp.concatenate([xi, xi[:, : SEQ_PAD - SEQ]], axis=1)
    return _embed_sc(x_pad.reshape(BATCH * SEQ_PAD), W)


# speedup vs baseline: 10.4290x; 1.8061x over previous
import functools

import jax
import jax.numpy as jnp
from jax import lax
from jax.experimental import pallas as pl
from jax.experimental.pallas import tpu as pltpu
from jax.experimental.pallas import tpu_sc as plsc

D_MODEL = 128
SEQ = 50
BATCH = 4096
B_TOTAL = BATCH * SEQ
NUM_CORES = 2
NUM_SUBCORES = 16
NW = NUM_CORES * NUM_SUBCORES
BPW = B_TOTAL // NW
CHUNK = 128
NCHUNK = BPW // CHUNK
NB = 5

_mesh = plsc.VectorSubcoreMesh(core_axis_name="c", subcore_axis_name="s")


@functools.partial(
    pl.kernel,
    mesh=_mesh,
    out_type=jax.ShapeDtypeStruct((B_TOTAL, D_MODEL), jnp.float32),
    scratch_types=[
        pltpu.VMEM((BPW,), jnp.int32),
        pltpu.VMEM((NB, CHUNK, D_MODEL), jnp.float32),
        pltpu.SemaphoreType.DMA((NB,)),
    ],
    compiler_params=pltpu.CompilerParams(use_tc_tiling_on_sc=True),
)
def _embed_sc(x_hbm, w_hbm, out_hbm, idx_v, rows_v, sems):
    wid = lax.axis_index("s") * NUM_CORES + lax.axis_index("c")
    base = wid * BPW
    pltpu.sync_copy(x_hbm.at[pl.ds(base, BPW)], idx_v)

    def gather(c, b):
        return pltpu.make_async_copy(
            w_hbm.at[idx_v.at[pl.ds(c * CHUNK, CHUNK)]],
            rows_v.at[b], sems.at[b])

    for b in range(NB):
        gather(b, b).start()

    def body(t, carry):
        c0 = t * NB
        for b in range(NB):
            c = c0 + b
            gather(c, b).wait()
            pltpu.sync_copy(rows_v.at[b],
                            out_hbm.at[pl.ds(base + c * CHUNK, CHUNK)])

            @pl.when(c + NB < NCHUNK)
            def _():
                gather(c + NB, b).start()

        return carry

    lax.fori_loop(0, NCHUNK // NB, body, 0)


def kernel(x, W):
    idx_flat = x.T.reshape(B_TOTAL).astype(jnp.int32)
    out = _embed_sc(idx_flat, W)
    return out.reshape(SEQ, BATCH, D_MODEL).transpose(1, 0, 2)
